# baseline (speedup 1.0000x reference)
import jax
import jax.numpy as jnp
from jax.experimental import pallas as pl

_NUM_SYM = 14
_FFT = 4096
_N = 2
_BATCH = 128
_TIN = 384
_TOUT = 448
_TS = 32

_BB = 16


def _body(x_ref, p_ref, o_ref):
    o_ref[:, 0:2 * _TS] = x_ref[:, 0:2 * _TS]
    o_ref[:, 3 * _TS:11 * _TS] = x_ref[:, 2 * _TS:10 * _TS]
    o_ref[:, 12 * _TS:14 * _TS] = x_ref[:, 10 * _TS:12 * _TS]
    p = p_ref[...]
    p0 = jnp.concatenate([p[0:_TS], p[0:_TS]], axis=1)
    p1 = jnp.concatenate([p[_TS:2 * _TS], p[_TS:2 * _TS]], axis=1)
    o_ref[:, 2 * _TS:3 * _TS] = jnp.broadcast_to(p0[None], (_BB, _TS, 256))
    o_ref[:, 11 * _TS:12 * _TS] = jnp.broadcast_to(p1[None], (_BB, _TS, 256))


def kernel(inputs, pilots):
    b = inputs.shape[0]
    x = inputs.reshape(b, _TIN, 128, _N).transpose(0, 1, 3, 2).reshape(b, _TIN, _N * 128)
    p = pilots.reshape(64, 128)
    out = pl.pallas_call(
        _body,
        grid=(b // _BB,),
        in_specs=[
            pl.BlockSpec((_BB, _TIN, _N * 128), lambda i: (i, 0, 0)),
            pl.BlockSpec((64, 128), lambda i: (0, 0)),
        ],
        out_specs=pl.BlockSpec((_BB, _TOUT, _N * 128), lambda i: (i, 0, 0)),
        out_shape=jax.ShapeDtypeStruct((b, _TOUT, _N * 128), inputs.dtype),
    )(x, p)
    return (out.reshape(b, _TOUT, _N, 128)
               .transpose(0, 1, 3, 2)
               .reshape(b, 1, 1, _NUM_SYM, _FFT, _N))
